# baseline (speedup 1.0000x reference)
import functools

import jax
import jax.numpy as jnp
from jax import lax
from jax.experimental import pallas as pl
from jax.experimental.pallas import tpu as pltpu
from jax.experimental.pallas import tpu_sc as plsc

EMBED = 128
BATCH = 16384
NUM_CORES = 2
NUM_SUBCORES = 16
NUM_WORKERS = NUM_CORES * NUM_SUBCORES
ROWS_PER_WORKER = BATCH // NUM_WORKERS
CHUNK = 128
NUM_CHUNKS = ROWS_PER_WORKER // CHUNK


def _gather_body(ids_hbm, tables_hbm, out_hbm, idx_v, rows_v, gsem, wsem):
    wid = lax.axis_index("s") * NUM_CORES + lax.axis_index("c")
    base = wid * ROWS_PER_WORKER
    pltpu.sync_copy(ids_hbm.at[pl.ds(wid * NUM_CHUNKS, NUM_CHUNKS)], idx_v)
    for c in range(NUM_CHUNKS):
        pltpu.async_copy(
            tables_hbm.at[idx_v.at[c]], rows_v.at[pl.ds(c * CHUNK, CHUNK)], gsem
        )
    for c in range(NUM_CHUNKS):
        pltpu.make_async_copy(
            tables_hbm.at[idx_v.at[c]], rows_v.at[pl.ds(c * CHUNK, CHUNK)], gsem
        ).wait()
    pltpu.async_copy(rows_v, out_hbm.at[pl.ds(base, ROWS_PER_WORKER)], wsem).wait()


@jax.jit
def _run(ids2d, tables):
    mesh = plsc.VectorSubcoreMesh(core_axis_name="c", subcore_axis_name="s")
    f = pl.kernel(
        _gather_body,
        mesh=mesh,
        out_type=jax.ShapeDtypeStruct((BATCH, EMBED), jnp.float32),
        scratch_types=[
            pltpu.VMEM((NUM_CHUNKS, CHUNK), jnp.int32),
            pltpu.VMEM((ROWS_PER_WORKER, EMBED), jnp.float32),
            pltpu.SemaphoreType.DMA,
            pltpu.SemaphoreType.DMA,
        ],
    )
    return f(ids2d, tables)


def kernel(dataset_ids, tables):
    ids2d = dataset_ids.astype(jnp.int32).reshape(BATCH // CHUNK, CHUNK)
    return _run(ids2d, tables)


# speedup vs baseline: 5.1806x; 5.1806x over previous
import functools

import jax
import jax.numpy as jnp
from jax import lax
from jax.experimental import pallas as pl
from jax.experimental.pallas import tpu as pltpu
from jax.experimental.pallas import tpu_sc as plsc

EMBED = 128
BATCH = 16384
NUM_CORES = 2
NUM_SUBCORES = 16
NUM_WORKERS = NUM_CORES * NUM_SUBCORES
ROWS_PER_WORKER = BATCH // NUM_WORKERS
CHUNK = 128
NUM_CHUNKS = ROWS_PER_WORKER // CHUNK


def _gather_body(ids_hbm, tables_hbm, out_hbm, idx_v, rows_v, tab_sh, gsem, wsem):
    sid = lax.axis_index("s")
    wid = sid * NUM_CORES + lax.axis_index("c")
    base = wid * ROWS_PER_WORKER
    with jax.named_scope("stage"):
        @pl.when(sid == 0)
        def _():
            pltpu.sync_copy(tables_hbm, tab_sh)
        pltpu.sync_copy(ids_hbm.at[pl.ds(wid * NUM_CHUNKS, NUM_CHUNKS)], idx_v)
        plsc.subcore_barrier()
    with jax.named_scope("gather"):
        for c in range(NUM_CHUNKS):
            pltpu.async_copy(
                tab_sh.at[idx_v.at[c]], rows_v.at[pl.ds(c * CHUNK, CHUNK)], gsem
            )
        for c in range(NUM_CHUNKS):
            pltpu.make_async_copy(
                tab_sh.at[idx_v.at[c]], rows_v.at[pl.ds(c * CHUNK, CHUNK)], gsem
            ).wait()
    with jax.named_scope("write"):
        pltpu.async_copy(rows_v, out_hbm.at[pl.ds(base, ROWS_PER_WORKER)], wsem).wait()


@jax.jit
def _run(ids2d, tables):
    mesh = plsc.VectorSubcoreMesh(core_axis_name="c", subcore_axis_name="s")
    f = pl.kernel(
        _gather_body,
        mesh=mesh,
        out_type=jax.ShapeDtypeStruct((BATCH, EMBED), jnp.float32),
        scratch_types=[
            pltpu.VMEM((NUM_CHUNKS, CHUNK), jnp.int32),
            pltpu.VMEM((ROWS_PER_WORKER, EMBED), jnp.float32),
            pltpu.VMEM_SHARED((6, EMBED), jnp.float32),
            pltpu.SemaphoreType.DMA,
            pltpu.SemaphoreType.DMA,
        ],
    )
    return f(ids2d, tables)


def kernel(dataset_ids, tables):
    ids2d = dataset_ids.astype(jnp.int32).reshape(BATCH // CHUNK, CHUNK)
    return _run(ids2d, tables)


# speedup vs baseline: 5.4183x; 1.0459x over previous
import functools

import jax
import jax.numpy as jnp
from jax import lax
from jax.experimental import pallas as pl
from jax.experimental.pallas import tpu as pltpu
from jax.experimental.pallas import tpu_sc as plsc

EMBED = 128
BATCH = 16384
NUM_CORES = 2
NUM_SUBCORES = 16
NUM_WORKERS = NUM_CORES * NUM_SUBCORES
ROWS_PER_WORKER = BATCH // NUM_WORKERS
CHUNK = 128
NUM_CHUNKS = ROWS_PER_WORKER // CHUNK


def _gather_body(ids_hbm, tables_hbm, out_hbm, idx_v, rows_v, tab_sh, gsem, wsem):
    sid = lax.axis_index("s")
    wid = sid * NUM_CORES + lax.axis_index("c")
    base = wid * ROWS_PER_WORKER
    with jax.named_scope("stage"):
        @pl.when(sid == 0)
        def _():
            pltpu.sync_copy(tables_hbm, tab_sh)
        pltpu.sync_copy(ids_hbm.at[pl.ds(wid * NUM_CHUNKS, NUM_CHUNKS)], idx_v)
        plsc.subcore_barrier()
    with jax.named_scope("gather"):
        for c in range(NUM_CHUNKS):
            pltpu.async_copy(
                tab_sh.at[idx_v.at[c]], rows_v.at[pl.ds(c * CHUNK, CHUNK)], gsem
            )
        for c in range(NUM_CHUNKS):
            pltpu.make_async_copy(
                tab_sh.at[idx_v.at[c]], rows_v.at[pl.ds(c * CHUNK, CHUNK)], gsem
            ).wait()
            pltpu.async_copy(
                rows_v.at[pl.ds(c * CHUNK, CHUNK)],
                out_hbm.at[pl.ds(base + c * CHUNK, CHUNK)],
                wsem,
            )
    with jax.named_scope("write"):
        for c in range(NUM_CHUNKS):
            pltpu.make_async_copy(
                rows_v.at[pl.ds(c * CHUNK, CHUNK)],
                out_hbm.at[pl.ds(base + c * CHUNK, CHUNK)],
                wsem,
            ).wait()


@jax.jit
def _run(ids2d, tables):
    mesh = plsc.VectorSubcoreMesh(core_axis_name="c", subcore_axis_name="s")
    f = pl.kernel(
        _gather_body,
        mesh=mesh,
        out_type=jax.ShapeDtypeStruct((BATCH, EMBED), jnp.float32),
        scratch_types=[
            pltpu.VMEM((NUM_CHUNKS, CHUNK), jnp.int32),
            pltpu.VMEM((ROWS_PER_WORKER, EMBED), jnp.float32),
            pltpu.VMEM_SHARED((6, EMBED), jnp.float32),
            pltpu.SemaphoreType.DMA,
            pltpu.SemaphoreType.DMA,
        ],
    )
    return f(ids2d, tables)


def kernel(dataset_ids, tables):
    ids2d = dataset_ids.astype(jnp.int32).reshape(BATCH // CHUNK, CHUNK)
    return _run(ids2d, tables)
